# baseline (speedup 1.0000x reference)
import jax
import jax.numpy as jnp
from jax.experimental import pallas as pl
from jax.experimental.pallas import tpu as pltpu

_B, _T, _D_AUDIO = 16, 1500, 128
_D_MODEL = 1024
_N_START, _N_END = 4, 1
_T_OUT = _N_START + _T + _N_END
_ROWS = _B * _T_OUT
_CHUNK = 2408
_N_CHUNKS = _ROWS // _CHUNK

_EVENTS = []
for _cc in range(_N_CHUNKS):
    _r0 = _CHUNK * _cc
    _starts, _ends = [], []
    for _k in range(_B + 1):
        _loc = _T_OUT * _k - _r0
        if _k >= 1 and 0 <= _loc - 1 < _CHUNK:
            _ends.append(_loc - 1)
        if _k <= _B - 1 and -3 <= _loc < _CHUNK:
            assert 0 <= _loc and _loc + _N_START <= _CHUNK, "start group straddles chunk"
            _starts.append(_loc)
    _EVENTS.append((_starts, _ends))


def _body(start_ids_ref, end_ids_ref, x_ref, m_ref, w_ref, embed_ref,
          out_ref, emb_ref, sems):
    c = pl.program_id(0)

    @pl.when(c == 0)
    def _gather():
        copies = []
        for j in range(_N_START):
            cp = pltpu.make_async_copy(
                embed_ref.at[pl.ds(start_ids_ref[j], 1), :],
                emb_ref.at[pl.ds(j, 1), :],
                sems.at[j],
            )
            cp.start()
            copies.append(cp)
        for j in range(_N_END):
            cp = pltpu.make_async_copy(
                embed_ref.at[pl.ds(end_ids_ref[j], 1), :],
                emb_ref.at[pl.ds(_N_START + j, 1), :],
                sems.at[_N_START + j],
            )
            cp.start()
            copies.append(cp)
        for cp in copies:
            cp.wait()

    x = x_ref[...] * m_ref[...]
    out_ref[...] = jnp.dot(x, w_ref[...], preferred_element_type=jnp.float32)

    for cc, (starts, ends) in enumerate(_EVENTS):
        @pl.when(c == cc)
        def _emb(starts=starts, ends=ends):
            for loc in starts:
                out_ref[loc:loc + _N_START, :] = emb_ref[0:_N_START, :]
            for loc in ends:
                out_ref[loc:loc + 1, :] = emb_ref[_N_START:_N_START + 1, :]


def kernel(sounds, masks, start_token_ids, end_token_ids, embed_table, W_enc):
    x_p = jnp.pad(sounds, ((0, 0), (_N_START, _N_END), (0, 0)))
    x_p = x_p.reshape(_ROWS, _D_AUDIO)
    m_p = jnp.pad(masks[..., None], ((0, 0), (_N_START, _N_END), (0, 0)))
    m_p = m_p.reshape(_ROWS, 1)
    grid_spec = pltpu.PrefetchScalarGridSpec(
        num_scalar_prefetch=2,
        grid=(_N_CHUNKS,),
        in_specs=[
            pl.BlockSpec((_CHUNK, _D_AUDIO), lambda c, *_: (c, 0)),
            pl.BlockSpec((_CHUNK, 1), lambda c, *_: (c, 0)),
            pl.BlockSpec((_D_AUDIO, _D_MODEL), lambda c, *_: (0, 0)),
            pl.BlockSpec(memory_space=pltpu.MemorySpace.HBM),
        ],
        out_specs=pl.BlockSpec((_CHUNK, _D_MODEL), lambda c, *_: (c, 0)),
        scratch_shapes=[
            pltpu.VMEM((8, _D_MODEL), jnp.float32),
            pltpu.SemaphoreType.DMA((_N_START + _N_END,)),
        ],
    )
    out = pl.pallas_call(
        _body,
        grid_spec=grid_spec,
        out_shape=jax.ShapeDtypeStruct((_ROWS, _D_MODEL), jnp.float32),
    )(start_token_ids.astype(jnp.int32), end_token_ids.astype(jnp.int32),
      x_p, m_p, W_enc, embed_table)
    return out.reshape(_B, _T_OUT, _D_MODEL)


# speedup vs baseline: 1.4667x; 1.4667x over previous
import jax
import jax.numpy as jnp
from jax.experimental import pallas as pl
from jax.experimental.pallas import tpu as pltpu

_B, _T, _D_AUDIO = 16, 1500, 128
_D_MODEL = 1024
_N_START, _N_END = 4, 1
_T_OUT = _N_START + _T + _N_END
_T_PAD = _N_START + _T


def _copies(y_ref, out_ref, sems, slot, b):
    big = pltpu.make_async_copy(
        y_ref.at[slot, pl.ds(0, _T_PAD), :],
        out_ref.at[b, pl.ds(0, _T_PAD), :],
        sems.at[slot, 0],
    )
    last = pltpu.make_async_copy(
        y_ref.at[slot, pl.ds(_T_PAD, 1), :],
        out_ref.at[b, pl.ds(_T_PAD, 1), :],
        sems.at[slot, 1],
    )
    return big, last


def _body(start_ids_ref, end_ids_ref, sounds_ref, masks_ref, w_ref,
          embed_ref, out_ref, x_ref, y_ref, emb_ref, sems):
    b = pl.program_id(0)
    slot = jax.lax.rem(b, 2)

    @pl.when(b == 0)
    def _init():
        copies = []
        for j in range(_N_START):
            cp = pltpu.make_async_copy(
                embed_ref.at[pl.ds(start_ids_ref[j], 1), :],
                emb_ref.at[pl.ds(j, 1), :],
                sems.at[0, 2],
            )
            cp.start()
            copies.append(cp)
        for j in range(_N_END):
            cp = pltpu.make_async_copy(
                embed_ref.at[pl.ds(end_ids_ref[j], 1), :],
                emb_ref.at[pl.ds(_N_START + j, 1), :],
                sems.at[1, 2],
            )
            cp.start()
            copies.append(cp)
        x_ref[0:_N_START, :] = jnp.zeros((_N_START, _D_AUDIO), jnp.float32)
        for cp in copies:
            cp.wait()

    @pl.when(b >= 2)
    def _wait_prev():
        big, last = _copies(y_ref, out_ref, sems, slot, b)
        big.wait()
        last.wait()

    x_ref[_N_START:_T_PAD, :] = sounds_ref[0] * masks_ref[0, 0][:, None]
    y_ref[slot, 0:_T_PAD, :] = jnp.dot(
        x_ref[...], w_ref[...], preferred_element_type=jnp.float32)
    y_ref[slot, 0:_N_START, :] = emb_ref[0:_N_START, :]
    y_ref[slot, _T_PAD:_T_PAD + 1, :] = emb_ref[_N_START:_N_START + 1, :]

    big, last = _copies(y_ref, out_ref, sems, slot, b)
    big.start()
    last.start()

    @pl.when(b == _B - 1)
    def _drain():
        for s in (1 - slot, slot):
            wbig, wlast = _copies(y_ref, out_ref, sems, s, b)
            wbig.wait()
            wlast.wait()


def kernel(sounds, masks, start_token_ids, end_token_ids, embed_table, W_enc):
    masks3 = masks.reshape(_B, 1, _T)
    grid_spec = pltpu.PrefetchScalarGridSpec(
        num_scalar_prefetch=2,
        grid=(_B,),
        in_specs=[
            pl.BlockSpec((1, _T, _D_AUDIO), lambda b, *_: (b, 0, 0)),
            pl.BlockSpec((1, 1, _T), lambda b, *_: (b, 0, 0)),
            pl.BlockSpec((_D_AUDIO, _D_MODEL), lambda b, *_: (0, 0)),
            pl.BlockSpec(memory_space=pltpu.MemorySpace.HBM),
        ],
        out_specs=pl.BlockSpec(memory_space=pltpu.MemorySpace.HBM),
        scratch_shapes=[
            pltpu.VMEM((_T_PAD, _D_AUDIO), jnp.float32),
            pltpu.VMEM((2, _T_PAD + 8, _D_MODEL), jnp.float32),
            pltpu.VMEM((8, _D_MODEL), jnp.float32),
            pltpu.SemaphoreType.DMA((2, 3)),
        ],
    )
    return pl.pallas_call(
        _body,
        grid_spec=grid_spec,
        out_shape=jax.ShapeDtypeStruct((_B, _T_OUT, _D_MODEL), jnp.float32),
        compiler_params=pltpu.CompilerParams(
            dimension_semantics=("arbitrary",)),
    )(start_token_ids.astype(jnp.int32), end_token_ids.astype(jnp.int32),
      sounds, masks3, W_enc, embed_table)
